# baseline (speedup 1.0000x reference)
import functools

import jax
import jax.numpy as jnp
from jax import lax
from jax.experimental import pallas as pl
from jax.experimental.pallas import tpu as pltpu
from jax.experimental.pallas import tpu_sc as plsc

_PAD = 0
_NBUF = 4
_LANES = 16


def _make_kernel(b, s, v, d):
    nw = 32
    k = 128
    n = b * s
    per_w = n // nw
    rows_w = b // nw
    n_chunks = per_w // k
    ng = n_chunks // _NBUF
    grp = _NBUF * k
    full = s // _LANES
    tail = s - full * _LANES
    pad = per_w + _LANES
    assert per_w % k == 0 and n_chunks % _NBUF == 0 and rows_w * s == per_w

    mesh = plsc.VectorSubcoreMesh(core_axis_name="c", subcore_axis_name="s")

    @functools.partial(
        pl.kernel,
        mesh=mesh,
        compiler_params=pltpu.CompilerParams(needs_layout_passes=False),
        out_type=jax.ShapeDtypeStruct((n, d), jnp.float32),
        scratch_types=[
            pltpu.VMEM((pad,), jnp.int32),
            pltpu.VMEM((pad,), jnp.int32),
            pltpu.VMEM((_NBUF, k, d), jnp.float32),
            pltpu.VMEM_SHARED((v, d), jnp.float32),
        ] + [pltpu.SemaphoreType.DMA] * (1 + 2 * _NBUF),
    )
    def body(in_hbm, table_hbm, out_hbm, in_v, idx_v, rows_v, table_sh,
             sl, *sems):
        sg = sems[:_NBUF]
        ss = sems[_NBUF:]
        sid = lax.axis_index("s")
        wid = sid * 2 + lax.axis_index("c")
        base = wid * per_w

        @pl.when(sid == 0)
        def _():
            pltpu.sync_copy(table_hbm, table_sh)

        src = in_hbm.at[pl.ds(base, per_w)]
        dst = in_v.at[pl.ds(0, per_w)]
        pltpu.async_copy(src, dst, sl)
        pltpu.make_async_copy(src, dst, sl).wait()
        plsc.subcore_barrier()

        def block_positions(g, carry_in):
            roffs = [(4 * g + i) * s for i in range(4)]
            xs = [in_v[pl.ds(ro, _LANES)] for ro in roffs]
            fifteen = jnp.minimum(xs[0], 0) + (_LANES - 1)
            tm1 = jnp.minimum(xs[0], 0) + (tail - 1)
            carries = []
            prev_cs = [None] * 4
            for i in range(4):
                mi = jnp.minimum(jnp.abs(xs[i]), 1)
                cs = plsc.cumsum(mi)
                idx_v[pl.ds(roffs[i], _LANES)] = cs * mi
                prev_cs[i] = cs
                carries.append(cs.at[fifteen].get(mode="promise_in_bounds"))
            for t in range(1, full):
                for i in range(4):
                    off = roffs[i] + t * _LANES
                    x = in_v[pl.ds(off, _LANES)]
                    mi = jnp.minimum(jnp.abs(x), 1)
                    cs = plsc.cumsum(mi) + carries[i]
                    idx_v[pl.ds(off, _LANES)] = cs * mi
                    prev_cs[i] = cs
                    carries[i] = cs.at[fifteen].get(mode="promise_in_bounds")
            for i in range(4):
                off = roffs[i] + s - _LANES
                carry = prev_cs[i].at[tm1].get(mode="promise_in_bounds")
                x = in_v[pl.ds(off, _LANES)]
                mi = jnp.minimum(jnp.abs(x), 1)
                cs = plsc.cumsum(mi) + carry
                idx_v[pl.ds(off, _LANES)] = cs * mi
            return carry_in

        blk_elems = 4 * s

        def blocks_needed(groups_done):
            return jnp.minimum((groups_done * grp + blk_elems - 1) // blk_elems,
                               rows_w // 4)

        r0 = blocks_needed(1)
        lax.fori_loop(0, r0, block_positions, 0)

        def out_slice(c):
            return out_hbm.at[pl.ds(base + c * k, k)]

        def idx_slice(c):
            return idx_v.at[pl.ds(c * k, k)]

        def group(j, rows_done):
            c = j * _NBUF

            @pl.when(j > 0)
            def _():
                for bb in range(_NBUF):
                    pltpu.make_async_copy(rows_v.at[bb],
                                          out_slice(c - _NBUF + bb),
                                          ss[bb]).wait()
                    pltpu.async_copy(table_sh.at[idx_slice(c + bb)],
                                     rows_v.at[bb], sg[bb])

            @pl.when(j == 0)
            def _():
                for bb in range(_NBUF):
                    pltpu.async_copy(table_sh.at[idx_slice(c + bb)],
                                     rows_v.at[bb], sg[bb])

            for bb in range(_NBUF):
                pltpu.make_async_copy(table_sh.at[idx_slice(c + bb)],
                                      rows_v.at[bb], sg[bb]).wait()
                pltpu.async_copy(rows_v.at[bb], out_slice(c + bb), ss[bb])

            target = blocks_needed(j + 2)
            lax.fori_loop(rows_done, target, block_positions, 0)
            new_done = jnp.maximum(target, rows_done)
            return new_done

        lax.fori_loop(0, ng, group, r0)
        for bb in range(_NBUF):
            pltpu.make_async_copy(rows_v.at[bb], out_slice(bb), ss[bb]).wait()

    return body


def kernel(input, table):
    b, s = input.shape
    v, d = table.shape
    inp = input.astype(jnp.int32)
    out = _make_kernel(b, s, v, d)(inp.reshape(b * s), table)
    return out.reshape(b, s, d)


# speedup vs baseline: 1.0049x; 1.0049x over previous
import functools

import jax
import jax.numpy as jnp
from jax import lax
from jax.experimental import pallas as pl
from jax.experimental.pallas import tpu as pltpu
from jax.experimental.pallas import tpu_sc as plsc

_PAD = 0


def _pos_body(inp_ref, pos_ref):
    x = inp_ref[...]
    mask = x != _PAD
    mf = mask.astype(jnp.bfloat16)
    s = x.shape[1]
    r = lax.broadcasted_iota(jnp.int32, (s, s), 0)
    c = lax.broadcasted_iota(jnp.int32, (s, s), 1)
    tri = (r <= c).astype(jnp.bfloat16)
    pos_f = jnp.dot(mf, tri, preferred_element_type=jnp.float32)
    pos = pos_f.astype(jnp.int32)
    pos_ref[...] = jnp.where(mask, pos, _PAD)


def _positions(inp):
    b, s = inp.shape
    blk = b
    return pl.pallas_call(
        _pos_body,
        out_shape=jax.ShapeDtypeStruct((b, s), jnp.int32),
        grid=(b // blk,),
        in_specs=[pl.BlockSpec((blk, s), lambda i: (i, 0))],
        out_specs=pl.BlockSpec((blk, s), lambda i: (i, 0)),
    )(inp)


_NBUF = 8
_K = 64


def _make_gather(n, v, d):
    nw = 32
    k = _K
    per_w = n // nw
    n_chunks = per_w // k
    ng = n_chunks // _NBUF
    assert per_w % k == 0 and n_chunks % _NBUF == 0

    mesh = plsc.VectorSubcoreMesh(core_axis_name="c", subcore_axis_name="s")

    @functools.partial(
        pl.kernel,
        mesh=mesh,
        out_type=jax.ShapeDtypeStruct((n, d), jnp.float32),
        scratch_types=[
            pltpu.VMEM((n_chunks, k), jnp.int32),
            pltpu.VMEM((_NBUF, k, d), jnp.float32),
            pltpu.VMEM_SHARED((v, d), jnp.float32),
        ] + [pltpu.SemaphoreType.DMA] * (1 + 2 * _NBUF),
    )
    def gather(pos_hbm, table_hbm, out_hbm, idx_v, rows_v, table_sh,
               sl, *sems):
        sg = sems[:_NBUF]
        ss = sems[_NBUF:]
        sid = lax.axis_index("s")
        wid = sid * 2 + lax.axis_index("c")
        cbase = wid * n_chunks

        @pl.when(sid == 0)
        def _():
            pltpu.sync_copy(table_hbm, table_sh)

        pltpu.async_copy(pos_hbm.at[pl.ds(cbase, n_chunks)], idx_v, sl)
        pltpu.make_async_copy(pos_hbm.at[pl.ds(cbase, n_chunks)], idx_v,
                              sl).wait()
        plsc.subcore_barrier()

        def out_slice(c):
            return out_hbm.at[pl.ds((cbase + c) * k, k)]

        def body(j, carry):
            c = j * _NBUF

            @pl.when(j > 0)
            def _():
                for b in range(_NBUF):
                    pltpu.make_async_copy(rows_v.at[b], out_slice(c - _NBUF + b),
                                          ss[b]).wait()
                    pltpu.async_copy(table_sh.at[idx_v.at[c + b]],
                                     rows_v.at[b], sg[b])

            @pl.when(j == 0)
            def _():
                for b in range(_NBUF):
                    pltpu.async_copy(table_sh.at[idx_v.at[c + b]],
                                     rows_v.at[b], sg[b])

            for b in range(_NBUF):
                pltpu.make_async_copy(table_sh.at[idx_v.at[c + b]],
                                      rows_v.at[b], sg[b]).wait()
                pltpu.async_copy(rows_v.at[b], out_slice(c + b), ss[b])
            return carry

        lax.fori_loop(0, ng, body, 0)
        for b in range(_NBUF):
            pltpu.make_async_copy(rows_v.at[b], out_slice(b), ss[b]).wait()

    return gather


def kernel(input, table):
    b, s = input.shape
    v, d = table.shape
    inp = input.astype(jnp.int32)
    pos = _positions(inp)
    n = b * s
    out = _make_gather(n, v, d)(pos.reshape(n // _K, _K), table)
    return out.reshape(b, s, d)


# speedup vs baseline: 1.0180x; 1.0131x over previous
import functools

import jax
import jax.numpy as jnp
from jax import lax
from jax.experimental import pallas as pl
from jax.experimental.pallas import tpu as pltpu
from jax.experimental.pallas import tpu_sc as plsc

_PAD = 0


def _pos_body(inp_ref, pos_ref):
    x = inp_ref[...]
    mask = x != _PAD
    mf = mask.astype(jnp.bfloat16)
    s = x.shape[1]
    r = lax.broadcasted_iota(jnp.int32, (s, s), 0)
    c = lax.broadcasted_iota(jnp.int32, (s, s), 1)
    tri = (r <= c).astype(jnp.bfloat16)
    pos_f = jnp.dot(mf, tri, preferred_element_type=jnp.float32)
    pos = pos_f.astype(jnp.int32)
    pos_ref[...] = jnp.where(mask, pos, _PAD)


def _positions(inp):
    b, s = inp.shape
    blk = b
    return pl.pallas_call(
        _pos_body,
        out_shape=jax.ShapeDtypeStruct((b, s), jnp.int32),
        grid=(b // blk,),
        in_specs=[pl.BlockSpec((blk, s), lambda i: (i, 0))],
        out_specs=pl.BlockSpec((blk, s), lambda i: (i, 0)),
    )(inp)


_NBUF = 5


def _make_gather(n, v, d):
    nw = 32
    k = 128
    per_w = n // nw
    n_chunks = per_w // k
    ng = n_chunks // _NBUF
    assert per_w % k == 0 and n_chunks % _NBUF == 0

    mesh = plsc.VectorSubcoreMesh(core_axis_name="c", subcore_axis_name="s")

    @functools.partial(
        pl.kernel,
        mesh=mesh,
        out_type=jax.ShapeDtypeStruct((n, d), jnp.float32),
        scratch_types=[
            pltpu.VMEM((n_chunks, k), jnp.int32),
            pltpu.VMEM((_NBUF, k, d), jnp.float32),
            pltpu.VMEM_SHARED((v, d), jnp.float32),
        ] + [pltpu.SemaphoreType.DMA] * (1 + 2 * _NBUF),
    )
    def gather(pos_hbm, table_hbm, out_hbm, idx_v, rows_v, table_sh,
               sl, *sems):
        sg = sems[:_NBUF]
        ss = sems[_NBUF:]
        sid = lax.axis_index("s")
        wid = sid * 2 + lax.axis_index("c")
        cbase = wid * n_chunks

        @pl.when(sid == 0)
        def _():
            pltpu.sync_copy(table_hbm, table_sh)

        pltpu.async_copy(pos_hbm.at[pl.ds(cbase, n_chunks)], idx_v, sl)
        pltpu.make_async_copy(pos_hbm.at[pl.ds(cbase, n_chunks)], idx_v,
                              sl).wait()
        plsc.subcore_barrier()

        def out_slice(c):
            return out_hbm.at[pl.ds((cbase + c) * k, k)]

        def body(j, carry):
            c = j * _NBUF

            @pl.when(j > 0)
            def _():
                for b in range(_NBUF):
                    pltpu.make_async_copy(rows_v.at[b], out_slice(c - _NBUF + b),
                                          ss[b]).wait()
                    pltpu.async_copy(table_sh.at[idx_v.at[c + b]],
                                     rows_v.at[b], sg[b])

            @pl.when(j == 0)
            def _():
                for b in range(_NBUF):
                    pltpu.async_copy(table_sh.at[idx_v.at[c + b]],
                                     rows_v.at[b], sg[b])

            for b in range(_NBUF):
                pltpu.make_async_copy(table_sh.at[idx_v.at[c + b]],
                                      rows_v.at[b], sg[b]).wait()
                pltpu.async_copy(rows_v.at[b], out_slice(c + b), ss[b])
            return carry

        lax.fori_loop(0, ng, body, 0)
        for b in range(_NBUF):
            pltpu.make_async_copy(rows_v.at[b], out_slice(b), ss[b]).wait()

    return gather


def kernel(input, table):
    b, s = input.shape
    v, d = table.shape
    inp = input.astype(jnp.int32)
    pos = _positions(inp)
    n = b * s
    out = _make_gather(n, v, d)(pos.reshape(n // 128, 128), table)
    return out.reshape(b, s, d)
